# baseline (speedup 1.0000x reference)
import jax
import jax.numpy as jnp
from jax import lax
from jax.experimental import pallas as pl
from jax.experimental.pallas import tpu as pltpu
from jax.experimental.pallas import tpu_sc as plsc

N_NODES = 10000
N_EDGES = 320000
D = 128
NC, NS, L = 2, 16, 16
NW = NC * NS
EPW = N_EDGES // NW
CH = 80
NCHUNK = EPW // CH
NPAD = 10240
RPT = NPAD // NS
ZR = 80


def _sc_body(x_hbm, src_hbm, dst_hbm, w_hbm,
             pooled_hbm, deg_hbm,
             idx_s, idx_d, wv, rows, deg, acc, gsem, ssem):
    cid = lax.axis_index("c")
    sid = lax.axis_index("s")
    wid = cid * NS + sid

    zeros = jnp.zeros((L,), jnp.float32)
    lane = lax.iota(jnp.int32, L)

    @pl.loop(0, ZR)
    def _zero_rows(r):
        for j in range(D // L):
            rows[0, r, pl.ds(j * L, L)] = zeros

    @pl.loop(0, NPAD // L)
    def _zero_deg(i):
        deg[pl.ds(i * L, L)] = zeros

    for k in range(RPT // ZR):
        pltpu.sync_copy(rows.at[0], acc.at[pl.ds(sid * RPT + k * ZR, ZR)])

    @pl.when(cid == 0)
    def _init_x():
        for k in range(RPT // ZR):
            base = sid * RPT + k * ZR

            @pl.when(base + ZR <= N_NODES)
            def _cp():
                pltpu.sync_copy(x_hbm.at[pl.ds(base, ZR)], rows.at[0])
                pltpu.sync_copy(rows.at[0], acc.at[pl.ds(base, ZR)])

    plsc.subcore_barrier()

    def _load_idx(i, slot):
        base = wid * EPW + i * CH
        pltpu.sync_copy(src_hbm.at[pl.ds(base, CH)], idx_s.at[slot])
        pltpu.sync_copy(dst_hbm.at[pl.ds(base, CH)], idx_d.at[slot])
        pltpu.sync_copy(w_hbm.at[pl.ds(base, CH)], wv.at[slot])

    _load_idx(0, 0)
    pltpu.async_copy(x_hbm.at[idx_s.at[0]], rows.at[0], gsem)

    @pl.loop(0, NCHUNK)
    def _chunk(i):
        b = lax.rem(i, 2)
        s = lax.rem(i, 3)
        pltpu.make_async_copy(x_hbm.at[pl.ds(0, CH)], rows.at[b], gsem).wait()

        @pl.when(i >= 1)
        def _wait_prev_scatter():
            pltpu.make_async_copy(x_hbm.at[pl.ds(0, CH)], rows.at[1 - b], ssem).wait()

        @pl.when(i + 1 < NCHUNK)
        def _prefetch():
            snext = lax.rem(i + 1, 3)
            _load_idx(i + 1, snext)
            pltpu.async_copy(x_hbm.at[idx_s.at[snext]], rows.at[1 - b], gsem)

        @pl.loop(0, CH // L)
        def _scale(g):
            vec = wv[s, pl.ds(g * L, L)]
            dvec = idx_d[s, pl.ds(g * L, L)]
            for r in range(L):
                e = g * L + r
                wb = jnp.full((L,), vec[r])
                for j in range(D // L):
                    sl = pl.ds(j * L, L)
                    rows[b, e, sl] = rows[b, e, sl] * wb
                d = dvec[r]
                deg[pl.ds(d, L)] = deg[pl.ds(d, L)] + jnp.where(lane == 0, wb, 0.0)

        pltpu.async_copy(rows.at[b], acc.at[idx_d.at[s]], ssem, add=True)

    pltpu.make_async_copy(x_hbm.at[pl.ds(0, CH)], rows.at[lax.rem(NCHUNK - 1, 2)],
                          ssem).wait()

    plsc.subcore_barrier()

    for k in range(RPT // ZR):
        base = sid * RPT + k * ZR
        pltpu.sync_copy(acc.at[pl.ds(base, ZR)], rows.at[0])
        pltpu.sync_copy(rows.at[0], pooled_hbm.at[cid, pl.ds(base, ZR)])
    pltpu.sync_copy(deg, deg_hbm.at[pl.ds(wid * NPAD, NPAD)])


BLK = 1024


def _tc_body(pooled_ref, deg_ref, o_ref):
    p = pooled_ref[0] + pooled_ref[1]
    dtot = jnp.sum(deg_ref[...], axis=0)
    o_ref[...] = p * (1.0 / (1.0 + dtot))[:, None]


def kernel(x, edge_weight, edge_index):
    x = x.astype(jnp.float32)
    w = jnp.squeeze(edge_weight, -1).astype(jnp.float32)
    src = edge_index[0].astype(jnp.int32)
    dst = edge_index[1].astype(jnp.int32)

    sc = pl.kernel(
        _sc_body,
        out_type=[jax.ShapeDtypeStruct((NC, NPAD, D), jnp.float32),
                  jax.ShapeDtypeStruct((NW * NPAD,), jnp.float32)],
        mesh=plsc.VectorSubcoreMesh(core_axis_name="c", subcore_axis_name="s",
                                    num_cores=NC, num_subcores=NS),
        scratch_types=[
            pltpu.VMEM((3, CH), jnp.int32),
            pltpu.VMEM((3, CH), jnp.int32),
            pltpu.VMEM((3, CH), jnp.float32),
            pltpu.VMEM((2, CH, D), jnp.float32),
            pltpu.VMEM((NPAD,), jnp.float32),
            pltpu.VMEM_SHARED((NPAD, D), jnp.float32),
            pltpu.SemaphoreType.DMA,
            pltpu.SemaphoreType.DMA,
        ],
    )
    pooled, deg = sc(x, src, dst, w)
    deg = deg.reshape(NW, NPAD)

    out = pl.pallas_call(
        _tc_body,
        grid=(NPAD // BLK,),
        in_specs=[pl.BlockSpec((NC, BLK, D), lambda i: (0, i, 0)),
                  pl.BlockSpec((NW, BLK), lambda i: (0, i))],
        out_specs=pl.BlockSpec((BLK, D), lambda i: (i, 0)),
        out_shape=jax.ShapeDtypeStruct((NPAD, D), jnp.float32),
    )(pooled, deg)
    return out[:N_NODES]


# speedup vs baseline: 1.8255x; 1.8255x over previous
import jax
import jax.numpy as jnp
from jax import lax
from jax.experimental import pallas as pl
from jax.experimental.pallas import tpu as pltpu
from jax.experimental.pallas import tpu_sc as plsc

N_NODES = 10000
N_EDGES = 320000
D = 128
NC, NS, L = 2, 16, 16
NW = NC * NS
EPW = N_EDGES // NW
CH = 80
NCHUNK = EPW // CH
NPAD = 10240
RPT = NPAD // NS
ZR = 80


def _sc_body(x_hbm, src_hbm, dst_hbm, w_hbm,
             pooled_hbm, deg_hbm,
             idx_s, idx_d, wv, rows, deg, acc, gsem, ssem):
    cid = lax.axis_index("c")
    sid = lax.axis_index("s")
    wid = cid * NS + sid

    zeros = jnp.zeros((L,), jnp.float32)
    lane = lax.iota(jnp.int32, L)

    @pl.loop(0, ZR)
    def _zero_rows(r):
        for j in range(D // L):
            rows[0, r, pl.ds(j * L, L)] = zeros

    @pl.loop(0, NPAD // L)
    def _zero_deg(i):
        deg[pl.ds(i * L, L)] = zeros

    for k in range(RPT // ZR):
        pltpu.sync_copy(rows.at[0], acc.at[pl.ds(sid * RPT + k * ZR, ZR)])

    @pl.when(cid == 0)
    def _init_x():
        for k in range(RPT // ZR):
            base = sid * RPT + k * ZR

            @pl.when(base + ZR <= N_NODES)
            def _cp():
                pltpu.sync_copy(x_hbm.at[pl.ds(base, ZR)], rows.at[0])
                pltpu.sync_copy(rows.at[0], acc.at[pl.ds(base, ZR)])

    plsc.subcore_barrier()

    def _load_idx(i, slot):
        base = wid * EPW + i * CH
        pltpu.sync_copy(src_hbm.at[pl.ds(base, CH)], idx_s.at[slot])
        pltpu.sync_copy(dst_hbm.at[pl.ds(base, CH)], idx_d.at[slot])
        pltpu.sync_copy(w_hbm.at[pl.ds(base, CH)], wv.at[slot])

    def _scale(su, ru):
        @pl.loop(0, CH // L)
        def _grp(g):
            vec = wv[su, pl.ds(g * L, L)]
            dvec = idx_d[su, pl.ds(g * L, L)]
            for r in range(L):
                e = g * L + r
                wb = jnp.full((L,), vec[r])
                for j in range(D // L):
                    sl = pl.ds(j * L, L)
                    rows[ru, e, sl] = rows[ru, e, sl] * wb
                d = dvec[r]
                deg[pl.ds(d, L)] = deg[pl.ds(d, L)] + jnp.where(lane == 0, wb, 0.0)

    def _wait_dma(sem, ru):
        pltpu.make_async_copy(x_hbm.at[pl.ds(0, CH)], rows.at[ru], sem).wait()

    def _emit_chunk(i, u, prefetch, wait_prev):
        _wait_dma(gsem, u)
        if prefetch:
            nu = (u + 1) % 3
            _load_idx(i + 1, nu)
            pltpu.async_copy(x_hbm.at[idx_s.at[nu]], rows.at[nu], gsem)
        _scale(u, u)
        if wait_prev is not None:
            wait_prev()
        pltpu.async_copy(rows.at[u], acc.at[idx_d.at[u]], ssem, add=True)

    _load_idx(0, 0)
    pltpu.async_copy(x_hbm.at[idx_s.at[0]], rows.at[0], gsem)

    NTRIP = (NCHUNK - 2) // 3

    @pl.loop(0, NTRIP)
    def _triple(p):
        i0 = p * 3
        for u in range(3):
            if u == 0:
                def _w(p=p, u=u):
                    @pl.when(p >= 1)
                    def _():
                        _wait_dma(ssem, 2)
                wait_prev = _w
            else:
                wait_prev = lambda u=u: _wait_dma(ssem, u - 1)
            _emit_chunk(i0 + u, u, prefetch=True, wait_prev=wait_prev)

    for i in range(3 * NTRIP, NCHUNK):
        u = i % 3
        wp = (lambda u=u: _wait_dma(ssem, (u - 1) % 3)) if i >= 1 else None
        _emit_chunk(i, u, prefetch=(i + 1 < NCHUNK), wait_prev=wp)
    _wait_dma(ssem, (NCHUNK - 1) % 3)

    plsc.subcore_barrier()

    for k in range(RPT // ZR):
        base = sid * RPT + k * ZR
        pltpu.sync_copy(acc.at[pl.ds(base, ZR)], rows.at[0])
        pltpu.sync_copy(rows.at[0], pooled_hbm.at[cid, pl.ds(base, ZR)])
    pltpu.sync_copy(deg, deg_hbm.at[pl.ds(wid * NPAD, NPAD)])


BLK = 1024


def _tc_body(pooled_ref, deg_ref, o_ref):
    p = pooled_ref[0] + pooled_ref[1]
    dtot = jnp.sum(deg_ref[...], axis=0)
    o_ref[...] = p * (1.0 / (1.0 + dtot))[:, None]


def kernel(x, edge_weight, edge_index):
    x = x.astype(jnp.float32)
    w = jnp.squeeze(edge_weight, -1).astype(jnp.float32)
    src = edge_index[0].astype(jnp.int32)
    dst = edge_index[1].astype(jnp.int32)

    sc = pl.kernel(
        _sc_body,
        out_type=[jax.ShapeDtypeStruct((NC, NPAD, D), jnp.float32),
                  jax.ShapeDtypeStruct((NW * NPAD,), jnp.float32)],
        mesh=plsc.VectorSubcoreMesh(core_axis_name="c", subcore_axis_name="s",
                                    num_cores=NC, num_subcores=NS),
        scratch_types=[
            pltpu.VMEM((3, CH), jnp.int32),
            pltpu.VMEM((3, CH), jnp.int32),
            pltpu.VMEM((3, CH), jnp.float32),
            pltpu.VMEM((3, CH, D), jnp.float32),
            pltpu.VMEM((NPAD,), jnp.float32),
            pltpu.VMEM_SHARED((NPAD, D), jnp.float32),
            pltpu.SemaphoreType.DMA,
            pltpu.SemaphoreType.DMA,
        ],
    )
    pooled, deg = sc(x, src, dst, w)
    deg = deg.reshape(NW, NPAD)

    out = pl.pallas_call(
        _tc_body,
        grid=(NPAD // BLK,),
        in_specs=[pl.BlockSpec((NC, BLK, D), lambda i: (0, i, 0)),
                  pl.BlockSpec((NW, BLK), lambda i: (0, i))],
        out_specs=pl.BlockSpec((BLK, D), lambda i: (i, 0)),
        out_shape=jax.ShapeDtypeStruct((NPAD, D), jnp.float32),
    )(pooled, deg)
    return out[:N_NODES]


# speedup vs baseline: 2.2796x; 1.2488x over previous
import jax
import jax.numpy as jnp
from jax import lax
from jax.experimental import pallas as pl
from jax.experimental.pallas import tpu as pltpu
from jax.experimental.pallas import tpu_sc as plsc

N_NODES = 10000
N_EDGES = 320000
D = 128
NC, NS, L = 2, 16, 16
NW = NC * NS
EPW = N_EDGES // NW
CH = 80
NCHUNK = EPW // CH
NPAD = 10240
RPT = NPAD // NS
ZR = 80


def _sc_body(x_hbm, pki_hbm, wch_hbm,
             pooled_hbm, deg_hbm,
             pkv, wv, rows, deg, acc, gsem, ssem, isem):
    cid = lax.axis_index("c")
    sid = lax.axis_index("s")
    wid = cid * NS + sid

    zeros = jnp.zeros((L,), jnp.float32)
    lane = lax.iota(jnp.int32, L)

    @pl.loop(0, ZR)
    def _zero_rows(r):
        for j in range(D // L):
            rows[0, r, pl.ds(j * L, L)] = zeros

    @pl.loop(0, NPAD // L)
    def _zero_deg(i):
        deg[pl.ds(i * L, L)] = zeros

    for k in range(RPT // ZR):
        pltpu.sync_copy(rows.at[0], acc.at[pl.ds(sid * RPT + k * ZR, ZR)])

    @pl.when(cid == 0)
    def _init_x():
        for k in range(RPT // ZR):
            base = sid * RPT + k * ZR

            @pl.when(base + ZR <= N_NODES)
            def _cp():
                pltpu.sync_copy(x_hbm.at[pl.ds(base, ZR)], rows.at[0])
                pltpu.sync_copy(rows.at[0], acc.at[pl.ds(base, ZR)])

    plsc.subcore_barrier()

    def _load_idx(i, slot):
        pltpu.async_copy(pki_hbm.at[wid, i], pkv.at[slot], isem)
        pltpu.async_copy(wch_hbm.at[wid, i], wv.at[slot], isem)

    def _wait_idx(i, slot):
        pltpu.make_async_copy(pki_hbm.at[wid, i], pkv.at[slot], isem).wait()
        pltpu.make_async_copy(wch_hbm.at[wid, i], wv.at[slot], isem).wait()

    def _scale(su, ru):
        @pl.loop(0, CH // L)
        def _grp(g):
            vec = wv[su, 0, pl.ds(g * L, L)]
            dvec = pkv[su, 1, pl.ds(g * L, L)]
            for r in range(L):
                e = g * L + r
                wb = jnp.full((L,), vec[r])
                for j in range(D // L):
                    sl = pl.ds(j * L, L)
                    rows[ru, e, sl] = rows[ru, e, sl] * wb
                d = dvec[r]
                deg[pl.ds(d, L)] = deg[pl.ds(d, L)] + jnp.where(lane == 0, wb, 0.0)

    def _wait_dma(sem, ru):
        pltpu.make_async_copy(x_hbm.at[pl.ds(0, CH)], rows.at[ru], sem).wait()

    def _emit_chunk(i, u, prefetch, wait_prev):
        nu = (u + 1) % 3
        if prefetch:
            _load_idx(i + 1, nu)
        _wait_dma(gsem, u)
        if prefetch:
            _wait_idx(i + 1, nu)
            pltpu.async_copy(x_hbm.at[pkv.at[nu, 0]], rows.at[nu], gsem)
        _scale(u, u)
        if wait_prev is not None:
            wait_prev()
        pltpu.async_copy(rows.at[u], acc.at[pkv.at[u, 1]], ssem, add=True)

    _load_idx(0, 0)
    _wait_idx(0, 0)
    pltpu.async_copy(x_hbm.at[pkv.at[0, 0]], rows.at[0], gsem)

    NTRIP = (NCHUNK - 2) // 3

    @pl.loop(0, NTRIP)
    def _triple(p):
        i0 = p * 3
        for u in range(3):
            if u == 0:
                def _w(p=p, u=u):
                    @pl.when(p >= 1)
                    def _():
                        _wait_dma(ssem, 2)
                wait_prev = _w
            else:
                wait_prev = lambda u=u: _wait_dma(ssem, u - 1)
            _emit_chunk(i0 + u, u, prefetch=True, wait_prev=wait_prev)

    for i in range(3 * NTRIP, NCHUNK):
        u = i % 3
        wp = (lambda u=u: _wait_dma(ssem, (u - 1) % 3)) if i >= 1 else None
        _emit_chunk(i, u, prefetch=(i + 1 < NCHUNK), wait_prev=wp)
    _wait_dma(ssem, (NCHUNK - 1) % 3)

    plsc.subcore_barrier()

    for k in range(RPT // ZR):
        base = sid * RPT + k * ZR
        pltpu.sync_copy(acc.at[pl.ds(base, ZR)], rows.at[0])
        pltpu.sync_copy(rows.at[0], pooled_hbm.at[cid, pl.ds(base, ZR)])
    pltpu.sync_copy(deg, deg_hbm.at[pl.ds(wid * NPAD, NPAD)])


BLK = 1024


def _tc_body(pooled_ref, deg_ref, o_ref):
    p = pooled_ref[0] + pooled_ref[1]
    dtot = jnp.sum(deg_ref[...], axis=0)
    o_ref[...] = p * (1.0 / (1.0 + dtot))[:, None]


def kernel(x, edge_weight, edge_index):
    x = x.astype(jnp.float32)
    w = jnp.squeeze(edge_weight, -1).astype(jnp.float32)
    src = edge_index[0].astype(jnp.int32)
    dst = edge_index[1].astype(jnp.int32)
    pki = jnp.stack([src.reshape(NW, NCHUNK, CH),
                     dst.reshape(NW, NCHUNK, CH)], axis=2)
    wch = w.reshape(NW, NCHUNK, 1, CH)

    sc = pl.kernel(
        _sc_body,
        out_type=[jax.ShapeDtypeStruct((NC, NPAD, D), jnp.float32),
                  jax.ShapeDtypeStruct((NW * NPAD,), jnp.float32)],
        mesh=plsc.VectorSubcoreMesh(core_axis_name="c", subcore_axis_name="s",
                                    num_cores=NC, num_subcores=NS),
        scratch_types=[
            pltpu.VMEM((3, 2, CH), jnp.int32),
            pltpu.VMEM((3, 1, CH), jnp.float32),
            pltpu.VMEM((3, CH, D), jnp.float32),
            pltpu.VMEM((NPAD,), jnp.float32),
            pltpu.VMEM_SHARED((NPAD, D), jnp.float32),
            pltpu.SemaphoreType.DMA,
            pltpu.SemaphoreType.DMA,
            pltpu.SemaphoreType.DMA,
        ],
    )
    pooled, deg = sc(x, pki, wch)
    deg = deg.reshape(NW, NPAD)

    out = pl.pallas_call(
        _tc_body,
        grid=(NPAD // BLK,),
        in_specs=[pl.BlockSpec((NC, BLK, D), lambda i: (0, i, 0)),
                  pl.BlockSpec((NW, BLK), lambda i: (0, i))],
        out_specs=pl.BlockSpec((BLK, D), lambda i: (i, 0)),
        out_shape=jax.ShapeDtypeStruct((NPAD, D), jnp.float32),
    )(pooled, deg)
    return out[:N_NODES]
